# baseline (speedup 1.0000x reference)
import jax
import jax.numpy as jnp
from jax import lax
from jax.experimental import pallas as pl
from jax.experimental.pallas import tpu as pltpu
from jax.experimental.pallas import tpu_sc as plsc

B, C, W = 32, 16, 65536
N = 32768
L = 16
CH = 16384
NCHUNK = N // CH
ITERS = CH // L
U = 8


def _tec_body(data_hbm, param_hbm, out_hbm, tex, pbuf, obuf):
    nc = 2
    b = lax.axis_index("s") * nc + lax.axis_index("c")
    pltpu.sync_copy(param_hbm.at[b], pbuf)
    tex[pl.ds(W, L)] = jnp.zeros((L,), jnp.float32)

    def prescale(j, carry):
        off = j * (U * L)
        for u in range(U):
            s = pl.ds(off + u * L, L)
            pbuf[s] = pbuf[s] * float(W - 1)
        return carry

    lax.fori_loop(0, N // (U * L), prescale, 0)

    def chan(c, carry):
        pltpu.sync_copy(data_hbm.at[b, c], tex.at[pl.ds(0, W)])
        for h in range(NCHUNK):
            def body(j, carry2):
                base = h * CH + j * (U * L)
                for u in range(U):
                    x = pbuf[pl.ds(base + u * L, L)]
                    i0 = x.astype(jnp.int32)
                    w = x - i0.astype(jnp.float32)
                    g0 = plsc.load_gather(tex, [i0])
                    g1 = plsc.load_gather(tex, [i0 + 1])
                    obuf[pl.ds(j * (U * L) + u * L, L)] = (
                        g0 + w * (g1 - g0))
                return carry2

            lax.fori_loop(0, ITERS // U, body, 0)
            pltpu.sync_copy(obuf, out_hbm.at[b, c, pl.ds(h * CH, CH)])
        return carry

    lax.fori_loop(0, C, chan, 0)


def kernel(data, param):
    mesh = plsc.VectorSubcoreMesh(core_axis_name="c", subcore_axis_name="s")
    f = pl.kernel(
        _tec_body,
        out_type=jax.ShapeDtypeStruct((B, C, N), jnp.float32),
        mesh=mesh,
        compiler_params=pltpu.CompilerParams(needs_layout_passes=False),
        scratch_types=[
            pltpu.VMEM((W + L,), jnp.float32),
            pltpu.VMEM((N,), jnp.float32),
            pltpu.VMEM((CH,), jnp.float32),
        ],
    )
    return f(data, param)


# speedup vs baseline: 1.1333x; 1.1333x over previous
import jax
import jax.numpy as jnp
from jax import lax
from jax.experimental import pallas as pl
from jax.experimental.pallas import tpu as pltpu
from jax.experimental.pallas import tpu_sc as plsc

B, C, W = 32, 16, 65536
N = 32768
L = 16
CH = 16384
NCHUNK = N // CH
ITERS = CH // L
U = 8


def _tec_body(data_hbm, param_hbm, out_hbm, tex, pbuf, obuf):
    nc = 2
    b = lax.axis_index("s") * nc + lax.axis_index("c")
    pltpu.sync_copy(param_hbm.at[b], pbuf)
    tex[pl.ds(W, L)] = jnp.zeros((L,), jnp.float32)

    def prescale(j, carry):
        off = j * (U * L)
        for u in range(U):
            s = pl.ds(off + u * L, L)
            pbuf[s] = pbuf[s] * float(W - 1)
        return carry

    lax.fori_loop(0, N // (U * L), prescale, 0)

    def chan(c, carry):
        pltpu.sync_copy(data_hbm.at[b, c], tex.at[pl.ds(0, W)])
        for h in range(NCHUNK):
            def body(j, carry2):
                base = h * CH + j * (U * L)
                for u in range(U):
                    x = pbuf[pl.ds(base + u * L, L)]
                    i0 = x.astype(jnp.int32)
                    w = x - i0.astype(jnp.float32)
                    g0 = plsc.load_gather(tex, [i0])
                    g1 = plsc.load_gather(tex, [i0 + 1])
                    obuf[pl.ds(j * (U * L) + u * L, L)] = (
                        g0 * (1.0 - w) + g1 * w)
                return carry2

            lax.fori_loop(0, ITERS // U, body, 0)
            pltpu.sync_copy(obuf, out_hbm.at[b, c, pl.ds(h * CH, CH)])
        return carry

    lax.fori_loop(0, C, chan, 0)


def kernel(data, param):
    mesh = plsc.VectorSubcoreMesh(core_axis_name="c", subcore_axis_name="s")
    f = pl.kernel(
        _tec_body,
        out_type=jax.ShapeDtypeStruct((B, C, N), jnp.float32),
        mesh=mesh,
        compiler_params=pltpu.CompilerParams(needs_layout_passes=False),
        scratch_types=[
            pltpu.VMEM((W + L,), jnp.float32),
            pltpu.VMEM((N,), jnp.float32),
            pltpu.VMEM((CH,), jnp.float32),
        ],
    )
    return f(data, param)


# speedup vs baseline: 2.0576x; 1.8156x over previous
import jax
import jax.numpy as jnp
from jax import lax
from jax.experimental import pallas as pl
from jax.experimental.pallas import tpu as pltpu
from jax.experimental.pallas import tpu_sc as plsc

B, C, W = 32, 16, 65536
N = 32768
L = 16
CH = 16384
NCHUNK = N // CH
ITERS = CH // L
U = 8


def _tec_body(data_hbm, param_hbm, out_hbm, tex, pbuf, obuf):
    nc = 2
    b = lax.axis_index("s") * nc + lax.axis_index("c")
    pltpu.sync_copy(param_hbm.at[b], pbuf)
    tex[pl.ds(W, L)] = jnp.zeros((L,), jnp.float32)

    @plsc.parallel_loop(0, N // L, unroll=U)
    def _(j):
        s = pl.ds(j * L, L)
        pbuf[s] = pbuf[s] * float(W - 1)

    def chan(c, carry):
        pltpu.sync_copy(data_hbm.at[b, c], tex.at[pl.ds(0, W)])
        for h in range(NCHUNK):
            @plsc.parallel_loop(0, ITERS, unroll=U)
            def _(j):
                x = pbuf[pl.ds(h * CH + j * L, L)]
                i0 = x.astype(jnp.int32)
                w = x - i0.astype(jnp.float32)
                g0 = plsc.load_gather(tex, [i0])
                g1 = plsc.load_gather(tex, [i0 + 1])
                obuf[pl.ds(j * L, L)] = g0 * (1.0 - w) + g1 * w

            pltpu.sync_copy(obuf, out_hbm.at[b, c, pl.ds(h * CH, CH)])
        return carry

    lax.fori_loop(0, C, chan, 0)


def kernel(data, param):
    mesh = plsc.VectorSubcoreMesh(core_axis_name="c", subcore_axis_name="s")
    f = pl.kernel(
        _tec_body,
        out_type=jax.ShapeDtypeStruct((B, C, N), jnp.float32),
        mesh=mesh,
        compiler_params=pltpu.CompilerParams(needs_layout_passes=False),
        scratch_types=[
            pltpu.VMEM((W + L,), jnp.float32),
            pltpu.VMEM((N,), jnp.float32),
            pltpu.VMEM((CH,), jnp.float32),
        ],
    )
    return f(data, param)


# speedup vs baseline: 2.2856x; 1.1108x over previous
import jax
import jax.numpy as jnp
from jax import lax
from jax.experimental import pallas as pl
from jax.experimental.pallas import tpu as pltpu
from jax.experimental.pallas import tpu_sc as plsc

B, C, W = 32, 16, 65536
N = 32768
L = 16
CH = 8192
NCHUNK = N // CH
ITERS = CH // L
U = 8


def _tec_body(data_hbm, param_hbm, out_hbm, tex, pbuf, ob0, ob1, sem0, sem1):
    nc = 2
    b = lax.axis_index("s") * nc + lax.axis_index("c")
    obufs = (ob0, ob1)
    sems = (sem0, sem1)
    pltpu.sync_copy(param_hbm.at[b], pbuf)

    @plsc.parallel_loop(0, N // L, unroll=U)
    def _(j):
        s = pl.ds(j * L, L)
        pbuf[s] = pbuf[s] * float(W - 1)

    def chan(c, carry):
        pltpu.sync_copy(data_hbm.at[b, c], tex)
        pending = {}
        for h in range(NCHUNK):
            slot = h % 2
            dst = out_hbm.at[b, c, pl.ds(h * CH, CH)]
            if h >= 2:
                pending[slot].wait()
            else:
                @pl.when(c > 0)
                def _():
                    pltpu.make_async_copy(obufs[slot], dst, sems[slot]).wait()

            @plsc.parallel_loop(0, ITERS, unroll=U)
            def _(j):
                x = pbuf[pl.ds(h * CH + j * L, L)]
                i0 = x.astype(jnp.int32)
                w = x - i0.astype(jnp.float32)
                g0 = plsc.load_gather(tex, [i0])
                g1 = plsc.load_gather(tex, [i0 + 1])
                obufs[slot][pl.ds(j * L, L)] = g0 * (1.0 - w) + g1 * w

            cp = pltpu.make_async_copy(obufs[slot], dst, sems[slot])
            cp.start()
            pending[slot] = cp
        return carry

    lax.fori_loop(0, C, chan, 0)
    for slot in range(2):
        pltpu.make_async_copy(
            obufs[slot], out_hbm.at[b, 0, pl.ds(0, CH)], sems[slot]).wait()


def kernel(data, param):
    mesh = plsc.VectorSubcoreMesh(core_axis_name="c", subcore_axis_name="s")
    f = pl.kernel(
        _tec_body,
        out_type=jax.ShapeDtypeStruct((B, C, N), jnp.float32),
        mesh=mesh,
        compiler_params=pltpu.CompilerParams(needs_layout_passes=False),
        scratch_types=[
            pltpu.VMEM((W,), jnp.float32),
            pltpu.VMEM((N,), jnp.float32),
            pltpu.VMEM((CH,), jnp.float32),
            pltpu.VMEM((CH,), jnp.float32),
            pltpu.SemaphoreType.DMA,
            pltpu.SemaphoreType.DMA,
        ],
    )
    return f(data, param)


# speedup vs baseline: 2.2912x; 1.0025x over previous
import jax
import jax.numpy as jnp
from jax import lax
from jax.experimental import pallas as pl
from jax.experimental.pallas import tpu as pltpu
from jax.experimental.pallas import tpu_sc as plsc

B, C, W = 32, 16, 65536
N = 32768
L = 16
CH = 8192
NCHUNK = N // CH
ITERS = CH // L
U = 8


def _tec_body(data_hbm, param_hbm, out_hbm, tex, pbuf, ob0, ob1,
              sem0, sem1, tsem):
    nc = 2
    b = lax.axis_index("s") * nc + lax.axis_index("c")
    obufs = (ob0, ob1)
    sems = (sem0, sem1)
    pltpu.sync_copy(param_hbm.at[b], pbuf)

    @plsc.parallel_loop(0, N // L, unroll=U)
    def _(j):
        s = pl.ds(j * L, L)
        pbuf[s] = pbuf[s] * float(W - 1)

    def chan(c, carry):
        W2 = W // 2
        t0 = pltpu.make_async_copy(
            data_hbm.at[b, c, pl.ds(0, W2)], tex.at[pl.ds(0, W2)], tsem)
        t1 = pltpu.make_async_copy(
            data_hbm.at[b, c, pl.ds(W2, W2)], tex.at[pl.ds(W2, W2)], tsem)
        t0.start()
        t1.start()
        t0.wait()
        t1.wait()
        pending = {}
        for h in range(NCHUNK):
            slot = h % 2
            dst = out_hbm.at[b, c, pl.ds(h * CH, CH)]
            if h >= 2:
                pending[slot].wait()
            else:
                @pl.when(c > 0)
                def _():
                    pltpu.make_async_copy(obufs[slot], dst, sems[slot]).wait()

            @plsc.parallel_loop(0, ITERS, unroll=U)
            def _(j):
                x = pbuf[pl.ds(h * CH + j * L, L)]
                i0 = x.astype(jnp.int32)
                w = x - i0.astype(jnp.float32)
                g0 = plsc.load_gather(tex, [i0])
                g1 = plsc.load_gather(tex, [i0 + 1])
                obufs[slot][pl.ds(j * L, L)] = g0 * (1.0 - w) + g1 * w

            cp = pltpu.make_async_copy(obufs[slot], dst, sems[slot])
            cp.start()
            pending[slot] = cp
        return carry

    lax.fori_loop(0, C, chan, 0)
    for slot in range(2):
        pltpu.make_async_copy(
            obufs[slot], out_hbm.at[b, 0, pl.ds(0, CH)], sems[slot]).wait()


def kernel(data, param):
    mesh = plsc.VectorSubcoreMesh(core_axis_name="c", subcore_axis_name="s")
    f = pl.kernel(
        _tec_body,
        out_type=jax.ShapeDtypeStruct((B, C, N), jnp.float32),
        mesh=mesh,
        compiler_params=pltpu.CompilerParams(needs_layout_passes=False),
        scratch_types=[
            pltpu.VMEM((W,), jnp.float32),
            pltpu.VMEM((N,), jnp.float32),
            pltpu.VMEM((CH,), jnp.float32),
            pltpu.VMEM((CH,), jnp.float32),
            pltpu.SemaphoreType.DMA,
            pltpu.SemaphoreType.DMA,
            pltpu.SemaphoreType.DMA,
        ],
    )
    return f(data, param)
